# Initial kernel scaffold; baseline (speedup 1.0000x reference)
import jax
import jax.numpy as jnp
from jax.experimental import pallas as pl


def kernel(x, edgemat, W, b):
    raise NotImplementedError("write your pallas kernel here")


# speedup vs baseline: 4.3670x; 4.3670x over previous
import functools

import jax
import jax.numpy as jnp
from jax import lax
from jax.experimental import pallas as pl
from jax.experimental.pallas import tpu as pltpu
from jax.experimental.pallas import tpu_sc as plsc

_NW = 32


def _sc_gather(idx2d, table):
    nrows, lw = idx2d.shape
    d = table.shape[1]
    mesh = plsc.VectorSubcoreMesh(core_axis_name="c", subcore_axis_name="s")

    @functools.partial(
        pl.kernel,
        mesh=mesh,
        out_type=jax.ShapeDtypeStruct((nrows, lw, d), table.dtype),
        scratch_types=[
            pltpu.VMEM((lw,), jnp.int32),
            pltpu.VMEM((lw, d), table.dtype),
            pltpu.SemaphoreType.DMA,
        ],
    )
    def k(idx_hbm, table_hbm, out_hbm, idx_v, rows_v, sem):
        w = lax.axis_index("s") * 2 + lax.axis_index("c")
        nit = (nrows - w + _NW - 1) // _NW

        def body(i, carry):
            c = w + i * _NW
            pltpu.sync_copy(idx_hbm.at[c], idx_v)
            pltpu.async_copy(table_hbm.at[idx_v], rows_v, sem).wait()
            pltpu.sync_copy(rows_v, out_hbm.at[c])
            return carry

        lax.fori_loop(0, nit, body, 0)

    return k(idx2d, table)


def _tc_linear(f5, Wt, b2, tile):
    _, E, F = f5.shape
    out_f = Wt.shape[2]

    def body(fref, wref, bref, oref):
        f0 = fref[0]
        f1 = fref[1]
        f2 = fref[2]
        f3 = fref[3]
        f4 = fref[4]
        acc = jnp.dot(f0, wref[0], preferred_element_type=jnp.float32)
        acc += jnp.dot(f1 + f3, wref[1], preferred_element_type=jnp.float32)
        acc += jnp.dot(f2 + f4, wref[2], preferred_element_type=jnp.float32)
        acc += jnp.dot(jnp.abs(f1 - f3), wref[3], preferred_element_type=jnp.float32)
        acc += jnp.dot(jnp.abs(f2 - f4), wref[4], preferred_element_type=jnp.float32)
        oref[...] = acc + bref[...]

    return pl.pallas_call(
        body,
        grid=(E // tile,),
        in_specs=[
            pl.BlockSpec((5, tile, F), lambda i: (0, i, 0)),
            pl.BlockSpec((5, F, out_f), lambda i: (0, 0, 0)),
            pl.BlockSpec((1, out_f), lambda i: (0, 0)),
        ],
        out_specs=pl.BlockSpec((tile, out_f), lambda i: (i, 0)),
        out_shape=jax.ShapeDtypeStruct((E, out_f), jnp.float32),
    )(f5, Wt, b2)


def kernel(x, edgemat, W, b):
    _, F, E, _ = x.shape
    K = edgemat.shape[2]
    out_f = W.shape[0]
    xt = jnp.transpose(x[0, :, :, 0])
    idx2d = jnp.transpose(edgemat[0]).reshape(-1, 128)
    fgath = _sc_gather(idx2d, xt).reshape(K, E, F)
    Wt = jnp.transpose(W.reshape(out_f, K, F), (1, 2, 0))
    y = _tc_linear(fgath, Wt, b.reshape(1, -1), tile=512)
    return jnp.transpose(y)[None, :, :, None]
